# baseline (speedup 1.0000x reference)
import functools

import jax
import jax.numpy as jnp
from jax import lax
from jax.experimental import pallas as pl
from jax.experimental.pallas import tpu as pltpu
from jax.experimental.pallas import tpu_sc as plsc

NC, NS, L = 2, 16, 16
NW = NC * NS
B = 16384
BPW = B // NW
V = 100
VP = 112
D = 32

_mesh = plsc.VectorSubcoreMesh(core_axis_name="c", subcore_axis_name="s")


@functools.partial(
    pl.kernel,
    out_type=jax.ShapeDtypeStruct((B,), jnp.float32),
    mesh=_mesh,
    scratch_types=[
        pltpu.VMEM((BPW,), jnp.int32),
        pltpu.VMEM((V, D), jnp.float32),
        pltpu.VMEM((D,), jnp.float32),
        pltpu.VMEM((1,), jnp.float32),
        pltpu.VMEM((VP,), jnp.float32),
        pltpu.VMEM((BPW,), jnp.float32),
        pltpu.SemaphoreType.DMA,
        pltpu.SemaphoreType.DMA,
        pltpu.SemaphoreType.DMA,
    ],
    compiler_params=pltpu.CompilerParams(needs_layout_passes=False),
)
def _sc_lut_gather(idx_hbm, table_hbm, w_hbm, b_hbm, out_hbm,
                   idx_v, table_v, w_v, b_v, lut_v, out_v,
                   sem_idx, sem_par, sem_out):
    wid = lax.axis_index("s") * NC + lax.axis_index("c")
    base = wid * BPW
    NQ = 4
    QW = BPW // NQ

    cps_idx = [
        pltpu.make_async_copy(idx_hbm.at[pl.ds(base + q * QW, QW)],
                              idx_v.at[pl.ds(q * QW, QW)], sem_idx)
        for q in range(NQ)
    ]
    for cp in cps_idx:
        cp.start()
    cp_tab = pltpu.make_async_copy(table_hbm, table_v, sem_par)
    cp_tab.start()
    cp_w = pltpu.make_async_copy(w_hbm, w_v, sem_par)
    cp_w.start()
    cp_b = pltpu.make_async_copy(b_hbm, b_v, sem_par)
    cp_b.start()
    cp_tab.wait()
    cp_w.wait()
    cp_b.wait()

    iota = lax.broadcasted_iota(jnp.int32, (L,), 0)
    zero16 = jnp.zeros((L,), jnp.int32)
    bvec = plsc.load_gather(b_v, [zero16])
    nchunk = VP // L
    rowvs = tuple(jnp.minimum(iota + k * L, V - 1) for k in range(nchunk))

    def matvec_body(c, accs):
        colv = (iota + c) & (D - 1)
        wrot = plsc.load_gather(w_v, [colv])
        return tuple(
            accs[k] + plsc.load_gather(table_v, [rowvs[k], colv]) * wrot
            for k in range(nchunk))

    accs = lax.fori_loop(
        0, D, matvec_body,
        tuple(jnp.zeros((L,), jnp.float32) for _ in range(nchunk)))
    for k in range(nchunk):
        x = accs[k] + bvec
        lut_v[pl.ds(k * L, L)] = 1.0 / (1.0 + jnp.exp(-x))

    cps_out = []
    for q in range(NQ):
        cps_idx[q].wait()

        def gather_body(j, carry, q=q):
            off = q * QW + j * L
            iv = idx_v[pl.ds(off, L)]
            out_v[pl.ds(off, L)] = plsc.load_gather(lut_v, [iv])
            return carry

        lax.fori_loop(0, QW // L, gather_body, 0)
        cp = pltpu.make_async_copy(out_v.at[pl.ds(q * QW, QW)],
                                   out_hbm.at[pl.ds(base + q * QW, QW)],
                                   sem_out)
        cp.start()
        cps_out.append(cp)
    for cp in cps_out:
        cp.wait()


def kernel(inputs, embedding_table, dense_w, dense_b):
    idx = inputs.reshape(B).astype(jnp.int32)
    w = dense_w.reshape(D)
    out = _sc_lut_gather(idx, embedding_table, w, dense_b.astype(jnp.float32))
    return out.reshape(B, 1)


# speedup vs baseline: 1.0535x; 1.0535x over previous
import functools

import jax
import jax.numpy as jnp
from jax import lax
from jax.experimental import pallas as pl
from jax.experimental.pallas import tpu as pltpu
from jax.experimental.pallas import tpu_sc as plsc

NC, NS, L = 2, 16, 16
NW = NC * NS
B = 16384
BPW = B // NW
V = 100
VP = 112
D = 32
POFF_W = D * VP
POFF_B = POFF_W + D
PLEN = POFF_B + L

_mesh = plsc.VectorSubcoreMesh(core_axis_name="c", subcore_axis_name="s")


@functools.partial(
    pl.kernel,
    out_type=jax.ShapeDtypeStruct((B,), jnp.float32),
    mesh=_mesh,
    scratch_types=[
        pltpu.VMEM((BPW,), jnp.int32),
        pltpu.VMEM((PLEN,), jnp.float32),
        pltpu.VMEM((VP,), jnp.float32),
        pltpu.VMEM((BPW,), jnp.float32),
        pltpu.SemaphoreType.DMA,
        pltpu.SemaphoreType.DMA,
    ],
    compiler_params=pltpu.CompilerParams(needs_layout_passes=False),
)
def _sc_lut_gather(idx_hbm, params_hbm, out_hbm,
                   idx_v, params_v, lut_v, out_v, sem_idx, sem_par):
    wid = lax.axis_index("s") * NC + lax.axis_index("c")
    base = wid * BPW

    cp_idx = pltpu.make_async_copy(idx_hbm.at[pl.ds(base, BPW)], idx_v, sem_idx)
    cp_idx.start()
    cp_par = pltpu.make_async_copy(params_hbm, params_v, sem_par)
    cp_par.start()
    cp_par.wait()

    nchunk = VP // L
    zero16 = jnp.zeros((L,), jnp.int32)

    def matvec_body(c, accs):
        wc = plsc.load_gather(params_v, [zero16 + (POFF_W + c)])[0]
        return tuple(accs[k] + params_v[pl.ds(c * VP + k * L, L)] * wc
                     for k in range(nchunk))

    accs = lax.fori_loop(
        0, D, matvec_body,
        tuple(jnp.zeros((L,), jnp.float32) for _ in range(nchunk)))
    bb = params_v[pl.ds(POFF_B, L)][0]
    for k in range(nchunk):
        x = accs[k] + bb
        lut_v[pl.ds(k * L, L)] = 1.0 / (1.0 + jnp.exp(-x))

    cp_idx.wait()

    def gather_body(j, carry):
        off = j * L
        iv = idx_v[pl.ds(off, L)]
        out_v[pl.ds(off, L)] = plsc.load_gather(lut_v, [iv])
        return carry

    lax.fori_loop(0, BPW // L, gather_body, 0)

    pltpu.sync_copy(out_v, out_hbm.at[pl.ds(base, BPW)])


def kernel(inputs, embedding_table, dense_w, dense_b):
    idx = inputs.reshape(B).astype(jnp.int32)
    params = jnp.concatenate([
        jnp.pad(embedding_table.T, ((0, 0), (0, VP - V))).reshape(-1),
        dense_w.reshape(D),
        dense_b.astype(jnp.float32),
        jnp.zeros((L - 1,), jnp.float32),
    ])
    out = _sc_lut_gather(idx, params)
    return out.reshape(B, 1)
